# baseline (speedup 1.0000x reference)
import jax
import jax.numpy as jnp
from jax.experimental import pallas as pl
from jax.experimental.pallas import tpu as pltpu

_N_CB = 4
_K = 1024
_E = 32


def _rowsum32(r):
    t = r + pltpu.roll(r, 24, 1)
    t = t + pltpu.roll(r, 16, 1)
    t = t + pltpu.roll(r, 8, 1)
    t = t + pltpu.roll(t, 28, 1)
    t = t + pltpu.roll(t, 30, 1)
    t = t + pltpu.roll(t, 31, 1)
    return t[:, 0:1]


def _rvq_kernel(x_ref, cb_ref, out_ref):
    xb = x_ref[0]
    res = xb.T
    quant = jnp.zeros_like(res)
    tok = res.shape[0]
    for i in range(_N_CB):
        cb = cb_ref[i]
        a2 = _rowsum32(res * res)
        b2 = _rowsum32(cb * cb).reshape(1, _K)
        s = jnp.dot(res, cb.T, preferred_element_type=jnp.float32)
        d = a2 - 2.0 * s + b2
        m = jnp.min(d, axis=1, keepdims=True)
        iota = jax.lax.broadcasted_iota(jnp.int32, (tok, _K), 1)
        idx = jnp.min(jnp.where(d == m, iota, _K), axis=1)
        onehot = (iota == idx[:, None]).astype(jnp.float32)
        q = jnp.dot(onehot, cb, preferred_element_type=jnp.float32,
                    precision=jax.lax.Precision.HIGHEST)
        res = res - q
        quant = quant + q
    out_ref[0] = quant.T


def kernel(embeddings, codebooks):
    B, E, H, W = embeddings.shape
    T = H * W
    x = embeddings.reshape(B, E, T)
    out = pl.pallas_call(
        _rvq_kernel,
        grid=(B,),
        in_specs=[
            pl.BlockSpec((1, E, T), lambda b: (b, 0, 0)),
            pl.BlockSpec((_N_CB, _K, _E), lambda b: (0, 0, 0)),
        ],
        out_specs=pl.BlockSpec((1, E, T), lambda b: (b, 0, 0)),
        out_shape=jax.ShapeDtypeStruct((B, E, T), jnp.float32),
        compiler_params=pltpu.CompilerParams(
            dimension_semantics=("parallel",)),
    )(x, codebooks)
    return out.reshape(B, E, H, W)


# speedup vs baseline: 1.5092x; 1.5092x over previous
import jax
import jax.numpy as jnp
from jax.experimental import pallas as pl
from jax.experimental.pallas import tpu as pltpu

_N_CB = 4
_K = 1024
_E = 32


def _rowsum32(r):
    t = r + pltpu.roll(r, 24, 1)
    t = t + pltpu.roll(r, 16, 1)
    t = t + pltpu.roll(r, 8, 1)
    t = t + pltpu.roll(t, 28, 1)
    t = t + pltpu.roll(t, 30, 1)
    t = t + pltpu.roll(t, 31, 1)
    return t[:, 0:1]


def _rvq_kernel(x_ref, cb_ref, out_ref):
    xb = x_ref[0]
    res = xb.T
    quant = jnp.zeros_like(res)
    tok = res.shape[0]
    for i in range(_N_CB):
        cb = cb_ref[i]
        a2 = _rowsum32(res * res)
        b2 = _rowsum32(cb * cb).reshape(1, _K)
        s = jnp.dot(res, cb.T, preferred_element_type=jnp.float32)
        d = a2 - 2.0 * s + b2
        m = jnp.min(d, axis=1, keepdims=True)
        iota = jax.lax.broadcasted_iota(jnp.int32, (tok, _K), 1)
        idx = jnp.min(jnp.where(d == m, iota, _K), axis=1)
        onehot = (iota == idx[:, None]).astype(jnp.float32)
        oh16 = onehot.astype(jnp.bfloat16)
        hi = cb.astype(jnp.bfloat16)
        r1 = cb - hi.astype(jnp.float32)
        mid = r1.astype(jnp.bfloat16)
        lo = (r1 - mid.astype(jnp.float32)).astype(jnp.bfloat16)
        q_hi = jnp.dot(oh16, hi, preferred_element_type=jnp.float32)
        q_mid = jnp.dot(oh16, mid, preferred_element_type=jnp.float32)
        q_lo = jnp.dot(oh16, lo, preferred_element_type=jnp.float32)
        q = (q_hi + q_mid) + q_lo
        res = res - q
        quant = quant + q
    out_ref[0] = quant.T


def kernel(embeddings, codebooks):
    B, E, H, W = embeddings.shape
    T = H * W
    x = embeddings.reshape(B, E, T)
    out = pl.pallas_call(
        _rvq_kernel,
        grid=(B,),
        in_specs=[
            pl.BlockSpec((1, E, T), lambda b: (b, 0, 0)),
            pl.BlockSpec((_N_CB, _K, _E), lambda b: (0, 0, 0)),
        ],
        out_specs=pl.BlockSpec((1, E, T), lambda b: (b, 0, 0)),
        out_shape=jax.ShapeDtypeStruct((B, E, T), jnp.float32),
        compiler_params=pltpu.CompilerParams(
            dimension_semantics=("parallel",)),
    )(x, codebooks)
    return out.reshape(B, E, H, W)


# speedup vs baseline: 2.5635x; 1.6985x over previous
import jax
import jax.numpy as jnp
from jax.experimental import pallas as pl
from jax.experimental.pallas import tpu as pltpu

_N_CB = 4
_K = 1024
_E = 32


def _rowsum32(r):
    t = r + pltpu.roll(r, 24, 1)
    t = t + pltpu.roll(r, 16, 1)
    t = t + pltpu.roll(r, 8, 1)
    t = t + pltpu.roll(t, 28, 1)
    t = t + pltpu.roll(t, 30, 1)
    t = t + pltpu.roll(t, 31, 1)
    return t[:, 0:1]


def _rvq_kernel(x_ref, cb_ref, cbs_ref, out_ref):
    xb = x_ref[0]
    res = xb.T
    quant = jnp.zeros_like(res)
    tok = res.shape[0]
    for i in range(_N_CB):
        cb = cb_ref[i]
        a2 = _rowsum32(res * res)
        b2 = _rowsum32(cb * cb).reshape(1, _K)
        s = jnp.dot(res, cb.T, preferred_element_type=jnp.float32)
        d = a2 - 2.0 * s + b2
        m = jnp.min(d, axis=1, keepdims=True)
        iota = jax.lax.broadcasted_iota(jnp.int32, (tok, _K), 1)
        idx = jnp.min(jnp.where(d == m, iota, _K), axis=1)
        onehot = (iota == idx[:, None]).astype(jnp.float32)
        oh16 = onehot.astype(jnp.bfloat16)
        qcat = jnp.dot(oh16, cbs_ref[i], preferred_element_type=jnp.float32)
        q = (qcat[:, :_E] + qcat[:, _E:2 * _E]) + qcat[:, 2 * _E:]
        res = res - q
        quant = quant + q
    out_ref[0] = quant.T


def kernel(embeddings, codebooks):
    B, E, H, W = embeddings.shape
    T = H * W
    x = embeddings.reshape(B, E, T)
    hi = codebooks.astype(jnp.bfloat16)
    r1 = codebooks - hi.astype(jnp.float32)
    mid = r1.astype(jnp.bfloat16)
    lo = (r1 - mid.astype(jnp.float32)).astype(jnp.bfloat16)
    cb_split = jnp.concatenate([hi, mid, lo], axis=2)
    out = pl.pallas_call(
        _rvq_kernel,
        grid=(B,),
        in_specs=[
            pl.BlockSpec((1, E, T), lambda b: (b, 0, 0)),
            pl.BlockSpec((_N_CB, _K, _E), lambda b: (0, 0, 0)),
            pl.BlockSpec((_N_CB, _K, 3 * _E), lambda b: (0, 0, 0)),
        ],
        out_specs=pl.BlockSpec((1, E, T), lambda b: (b, 0, 0)),
        out_shape=jax.ShapeDtypeStruct((B, E, T), jnp.float32),
        compiler_params=pltpu.CompilerParams(
            dimension_semantics=("parallel",)),
    )(x, codebooks, cb_split)
    return out.reshape(B, E, H, W)


# speedup vs baseline: 5.4650x; 2.1319x over previous
import jax
import jax.numpy as jnp
from jax.experimental import pallas as pl
from jax.experimental.pallas import tpu as pltpu

_N_CB = 4
_K = 1024
_E = 32


def _rowsum32(r):
    t = r + pltpu.roll(r, 24, 1)
    t = t + pltpu.roll(r, 16, 1)
    t = t + pltpu.roll(r, 8, 1)
    t = t + pltpu.roll(t, 28, 1)
    t = t + pltpu.roll(t, 30, 1)
    t = t + pltpu.roll(t, 31, 1)
    return t[:, 0:1]


def _rvq_kernel(x_ref, cb_ref, cbs_ref, out_ref):
    xb = x_ref[0]
    res = xb.T
    quant = jnp.zeros_like(res)
    tok = res.shape[0]
    for i in range(_N_CB):
        cb = cb_ref[i]
        a2 = jnp.sum(res * res, axis=1, keepdims=True)
        b2 = jnp.sum(cb * cb, axis=1)[None, :]
        s = jnp.dot(res, cb.T, preferred_element_type=jnp.float32)
        d = a2 - 2.0 * s + b2
        m = jnp.min(d, axis=1, keepdims=True)
        iota = jax.lax.broadcasted_iota(jnp.int32, (tok, _K), 1)
        idx = jnp.min(jnp.where(d == m, iota, _K), axis=1)
        onehot = (iota == idx[:, None]).astype(jnp.float32)
        oh16 = onehot.astype(jnp.bfloat16)
        qcat = jnp.dot(oh16, cbs_ref[i], preferred_element_type=jnp.float32)
        q = (qcat[:, :_E] + qcat[:, _E:2 * _E]) + qcat[:, 2 * _E:]
        res = res - q
        quant = quant + q
    out_ref[0] = quant.T


def kernel(embeddings, codebooks):
    B, E, H, W = embeddings.shape
    T = H * W
    x = embeddings.reshape(B, E, T)
    hi = codebooks.astype(jnp.bfloat16)
    r1 = codebooks - hi.astype(jnp.float32)
    mid = r1.astype(jnp.bfloat16)
    lo = (r1 - mid.astype(jnp.float32)).astype(jnp.bfloat16)
    cb_split = jnp.concatenate([hi, mid, lo], axis=2)
    out = pl.pallas_call(
        _rvq_kernel,
        grid=(B,),
        in_specs=[
            pl.BlockSpec((1, E, T), lambda b: (b, 0, 0)),
            pl.BlockSpec((_N_CB, _K, _E), lambda b: (0, 0, 0)),
            pl.BlockSpec((_N_CB, _K, 3 * _E), lambda b: (0, 0, 0)),
        ],
        out_specs=pl.BlockSpec((1, E, T), lambda b: (b, 0, 0)),
        out_shape=jax.ShapeDtypeStruct((B, E, T), jnp.float32),
        compiler_params=pltpu.CompilerParams(
            dimension_semantics=("parallel",)),
    )(x, codebooks, cb_split)
    return out.reshape(B, E, H, W)
